# baseline (speedup 1.0000x reference)
import functools

import jax
import jax.numpy as jnp
from jax import lax
from jax.experimental import pallas as pl
from jax.experimental.pallas import tpu as pltpu
from jax.experimental.pallas import tpu_sc as plsc

_N_RBF = 8
_R_CUT = 5.0
_C0 = 0.28209479177387814
_C1 = 0.4886025119029199

_LANES = 16
_SC_CHUNK = 2000


def _sc_gather_w(z, idx_i, idx_j, table_pad, n_nodes, n_edges):
    info = plsc.get_sparse_core_info()
    nc, ns = info.num_cores, info.num_subcores
    nw = nc * ns
    per_tile = n_edges // nw
    chunk = _SC_CHUNK
    n_chunks = per_tile // chunk
    mesh = plsc.VectorSubcoreMesh(core_axis_name="c", subcore_axis_name="s")

    @functools.partial(
        pl.kernel,
        mesh=mesh,
        out_type=jax.ShapeDtypeStruct((n_edges,), jnp.float32),
        scratch_types=[
            pltpu.VMEM((n_nodes,), jnp.int32),
            pltpu.VMEM((table_pad.shape[0],), jnp.float32),
            pltpu.VMEM((2, chunk), jnp.int32),
            pltpu.VMEM((2, chunk), jnp.int32),
            pltpu.VMEM((2, chunk), jnp.float32),
            pltpu.SemaphoreType.DMA,
            pltpu.SemaphoreType.DMA,
            pltpu.SemaphoreType.DMA,
            pltpu.SemaphoreType.DMA,
        ],
        compiler_params=pltpu.CompilerParams(
            needs_layout_passes=False,
            use_tc_tiling_on_sc=False,
        ),
    )
    def body(z_hbm, ii_hbm, jj_hbm, tab_hbm, w_hbm, z_v, tab_v, ii_v, jj_v,
             w_v, sem_i0, sem_i1, sem_o0, sem_o1):
        wid = lax.axis_index("s") * nc + lax.axis_index("c")
        base = wid * per_tile
        sems_in = (sem_i0, sem_i1)
        sems_out = (sem_o0, sem_o1)
        pltpu.sync_copy(z_hbm, z_v)
        pltpu.sync_copy(tab_hbm, tab_v)

        def start_in(c):
            b = c % 2
            off = base + c * chunk
            return (
                pltpu.async_copy(ii_hbm.at[pl.ds(off, chunk)], ii_v.at[b],
                                 sems_in[b]),
                pltpu.async_copy(jj_hbm.at[pl.ds(off, chunk)], jj_v.at[b],
                                 sems_in[b]),
            )

        in_flight = {0: start_in(0)}
        out_flight = {}
        for c in range(n_chunks):
            b = c % 2
            if c + 1 < n_chunks:
                in_flight[c + 1] = start_in(c + 1)
            for cp in in_flight.pop(c):
                cp.wait()
            if c >= 2:
                out_flight.pop(c - 2).wait()

            def vec_body(t):
                ii = ii_v[b, pl.ds(t, _LANES)]
                jj = jj_v[b, pl.ds(t, _LANES)]
                zi = plsc.load_gather(z_v, [ii])
                zj = plsc.load_gather(z_v, [jj])
                xi = plsc.load_gather(tab_v, [zi])
                xj = plsc.load_gather(tab_v, [zj])
                w_v[b, pl.ds(t, _LANES)] = xi * xj

            plsc.parallel_loop(0, chunk, _LANES, unroll=5)(vec_body)

            off = base + c * chunk
            out_flight[c] = pltpu.async_copy(
                w_v.at[b], w_hbm.at[pl.ds(off, chunk)], sems_out[b])
        for cp in out_flight.values():
            cp.wait()

    return body(z, idx_i, idx_j, table_pad)


def _tc_body(rx_ref, ry_ref, rz_ref, w_ref, out_ref):
    rx = rx_ref[...].reshape(1, -1)
    ry = ry_ref[...].reshape(1, -1)
    rz = rz_ref[...].reshape(1, -1)
    w = w_ref[...].reshape(1, -1)
    d2 = rx * rx + ry * ry + rz * rz
    inv_d = lax.rsqrt(d2)
    x = d2 * inv_d * (jnp.pi / _R_CUT)
    two_c = 2.0 * jnp.cos(x)
    t1 = jnp.sin(x) * (w * (jnp.sqrt(2.0 / _R_CUT)) * inv_d)
    rbf_rows = []
    s_prev, s_cur = jnp.zeros_like(t1), t1
    for _ in range(_N_RBF):
        rbf_rows.append(s_cur)
        s_prev, s_cur = s_cur, two_c * s_cur - s_prev
    rbf = jnp.concatenate(rbf_rows, axis=0)
    c1d = _C1 * inv_d
    out_ref[...] = jnp.concatenate(
        [_C0 * rbf, (c1d * ry) * rbf, (c1d * rz) * rbf, (c1d * rx) * rbf],
        axis=0,
    )


def kernel(z, idx_i, idx_j, r_ij, embed_table):
    n_edges = idx_i.shape[0]
    n_nodes = z.shape[0]
    table_flat = embed_table.reshape(-1)
    pad = (-table_flat.shape[0]) % 128
    table_pad = jnp.pad(table_flat, (0, pad))

    w = _sc_gather_w(z.astype(jnp.int32), idx_i.astype(jnp.int32),
                     idx_j.astype(jnp.int32), table_pad, n_nodes, n_edges)

    bl = 8192
    grid = -(-n_edges // bl)
    n_blk = grid * bl
    r_flat = jnp.pad(jnp.swapaxes(r_ij, 0, 1),
                     ((0, 0), (0, n_blk - n_edges))).reshape(-1)
    in_spec = pl.BlockSpec((bl,), lambda i: (i,))
    out_t = pl.pallas_call(
        _tc_body,
        grid=(grid,),
        in_specs=[
            pl.BlockSpec((bl,), lambda i: (i,)),
            pl.BlockSpec((bl,), lambda i: (grid + i,)),
            pl.BlockSpec((bl,), lambda i: (2 * grid + i,)),
            in_spec,
        ],
        out_specs=pl.BlockSpec((32, bl), lambda i: (0, i)),
        out_shape=jax.ShapeDtypeStruct((32, n_edges), jnp.float32),
        compiler_params=pltpu.CompilerParams(
            dimension_semantics=("arbitrary",),
        ),
    )(r_flat, r_flat, r_flat, w)
    return out_t.T


# speedup vs baseline: 2.6011x; 2.6011x over previous
import functools

import jax
import jax.numpy as jnp
from jax import lax
from jax.experimental import pallas as pl
from jax.experimental.pallas import tpu as pltpu
from jax.experimental.pallas import tpu_sc as plsc

_N_RBF = 8
_R_CUT = 5.0
_C0 = 0.28209479177387814
_C1 = 0.4886025119029199

_LANES = 16
_SC_CHUNK = 2000


def _sc_gather_w(z, idx_i, idx_j, table_pad, n_nodes, n_edges):
    info = plsc.get_sparse_core_info()
    nc, ns = info.num_cores, info.num_subcores
    nw = nc * ns
    per_tile = n_edges // nw
    chunk = _SC_CHUNK
    n_chunks = per_tile // chunk
    mesh = plsc.VectorSubcoreMesh(core_axis_name="c", subcore_axis_name="s")

    @functools.partial(
        pl.kernel,
        mesh=mesh,
        out_type=jax.ShapeDtypeStruct((n_edges,), jnp.float32),
        scratch_types=[
            pltpu.VMEM((n_nodes,), jnp.int32),
            pltpu.VMEM((table_pad.shape[0],), jnp.float32),
            pltpu.VMEM((2, chunk), jnp.int32),
            pltpu.VMEM((2, chunk), jnp.int32),
            pltpu.VMEM((2, chunk), jnp.float32),
            pltpu.SemaphoreType.DMA,
            pltpu.SemaphoreType.DMA,
            pltpu.SemaphoreType.DMA,
            pltpu.SemaphoreType.DMA,
        ],
        compiler_params=pltpu.CompilerParams(
            needs_layout_passes=False,
            use_tc_tiling_on_sc=False,
        ),
    )
    def body(z_hbm, ii_hbm, jj_hbm, tab_hbm, w_hbm, z_v, tab_v, ii_v, jj_v,
             w_v, sem_i0, sem_i1, sem_o0, sem_o1):
        wid = lax.axis_index("s") * nc + lax.axis_index("c")
        base = wid * per_tile
        sems_in = (sem_i0, sem_i1)
        sems_out = (sem_o0, sem_o1)
        pltpu.sync_copy(z_hbm, z_v)
        pltpu.sync_copy(tab_hbm, tab_v)

        def start_in(c):
            b = c % 2
            off = base + c * chunk
            return (
                pltpu.async_copy(ii_hbm.at[pl.ds(off, chunk)], ii_v.at[b],
                                 sems_in[b]),
                pltpu.async_copy(jj_hbm.at[pl.ds(off, chunk)], jj_v.at[b],
                                 sems_in[b]),
            )

        in_flight = {0: start_in(0)}
        out_flight = {}
        for c in range(n_chunks):
            b = c % 2
            if c + 1 < n_chunks:
                in_flight[c + 1] = start_in(c + 1)
            for cp in in_flight.pop(c):
                cp.wait()
            if c >= 2:
                out_flight.pop(c - 2).wait()

            def vec_body(t):
                ii = ii_v[b, pl.ds(t, _LANES)]
                jj = jj_v[b, pl.ds(t, _LANES)]
                zi = plsc.load_gather(z_v, [ii])
                zj = plsc.load_gather(z_v, [jj])
                xi = plsc.load_gather(tab_v, [zi])
                xj = plsc.load_gather(tab_v, [zj])
                w_v[b, pl.ds(t, _LANES)] = xi * xj

            plsc.parallel_loop(0, chunk, _LANES, unroll=5)(vec_body)

            off = base + c * chunk
            out_flight[c] = pltpu.async_copy(
                w_v.at[b], w_hbm.at[pl.ds(off, chunk)], sems_out[b])
        for cp in out_flight.values():
            cp.wait()

    return body(z, idx_i, idx_j, table_pad)


def _tc_body(rx_ref, ry_ref, rz_ref, w_ref, out_ref):
    rx = rx_ref[...].reshape(1, -1)
    ry = ry_ref[...].reshape(1, -1)
    rz = rz_ref[...].reshape(1, -1)
    w = w_ref[...].reshape(1, -1)
    d2 = rx * rx + ry * ry + rz * rz
    inv_d = lax.rsqrt(d2)
    x = d2 * inv_d * (jnp.pi / _R_CUT)
    two_c = 2.0 * jnp.cos(x)
    t1 = jnp.sin(x) * (w * (jnp.sqrt(2.0 / _R_CUT)) * inv_d)
    rbf_rows = []
    s_prev, s_cur = jnp.zeros_like(t1), t1
    for _ in range(_N_RBF):
        rbf_rows.append(s_cur)
        s_prev, s_cur = s_cur, two_c * s_cur - s_prev
    rbf = jnp.concatenate(rbf_rows, axis=0)
    c1d = _C1 * inv_d
    out_ref[...] = jnp.concatenate(
        [_C0 * rbf, (c1d * ry) * rbf, (c1d * rz) * rbf, (c1d * rx) * rbf],
        axis=0,
    )


def kernel(z, idx_i, idx_j, r_ij, embed_table):
    n_edges = idx_i.shape[0]
    n_nodes = z.shape[0]
    table_flat = embed_table.reshape(-1)
    pad = (-table_flat.shape[0]) % 128
    table_pad = jnp.pad(table_flat, (0, pad))

    w = _sc_gather_w(z.astype(jnp.int32), idx_i.astype(jnp.int32),
                     idx_j.astype(jnp.int32), table_pad, n_nodes, n_edges)

    bl = 16384
    grid = -(-n_edges // bl)
    in_spec = pl.BlockSpec((bl,), lambda i: (i,))
    out_t = pl.pallas_call(
        _tc_body,
        grid=(grid,),
        in_specs=[in_spec] * 4,
        out_specs=pl.BlockSpec((32, bl), lambda i: (0, i)),
        out_shape=jax.ShapeDtypeStruct((32, n_edges), jnp.float32),
        compiler_params=pltpu.CompilerParams(
            dimension_semantics=("arbitrary",),
        ),
    )(r_ij[:, 0], r_ij[:, 1], r_ij[:, 2], w)
    return out_t.T


# speedup vs baseline: 2.8991x; 1.1146x over previous
import functools

import jax
import jax.numpy as jnp
from jax import lax
from jax.experimental import pallas as pl
from jax.experimental.pallas import tpu as pltpu
from jax.experimental.pallas import tpu_sc as plsc

_N_RBF = 8
_R_CUT = 5.0
_C0 = 0.28209479177387814
_C1 = 0.4886025119029199

_LANES = 16
_SC_CHUNK = 2000


def _sc_gather_w(z, idx_i, idx_j, table_pad, n_nodes, n_edges):
    info = plsc.get_sparse_core_info()
    nc, ns = info.num_cores, info.num_subcores
    nw = nc * ns
    per_tile = n_edges // nw
    chunk = _SC_CHUNK
    n_chunks = per_tile // chunk
    mesh = plsc.VectorSubcoreMesh(core_axis_name="c", subcore_axis_name="s")

    @functools.partial(
        pl.kernel,
        mesh=mesh,
        out_type=jax.ShapeDtypeStruct((n_edges,), jnp.float32),
        scratch_types=[
            pltpu.VMEM((n_nodes,), jnp.int32),
            pltpu.VMEM((table_pad.shape[0],), jnp.float32),
            pltpu.VMEM((2, chunk), jnp.int32),
            pltpu.VMEM((2, chunk), jnp.int32),
            pltpu.VMEM((2, chunk), jnp.float32),
            pltpu.SemaphoreType.DMA,
            pltpu.SemaphoreType.DMA,
            pltpu.SemaphoreType.DMA,
            pltpu.SemaphoreType.DMA,
        ],
        compiler_params=pltpu.CompilerParams(
            needs_layout_passes=False,
            use_tc_tiling_on_sc=False,
        ),
    )
    def body(z_hbm, ii_hbm, jj_hbm, tab_hbm, w_hbm, z_v, tab_v, ii_v, jj_v,
             w_v, sem_i0, sem_i1, sem_o0, sem_o1):
        wid = lax.axis_index("s") * nc + lax.axis_index("c")
        base = wid * per_tile
        sems_in = (sem_i0, sem_i1)
        sems_out = (sem_o0, sem_o1)
        pltpu.sync_copy(z_hbm, z_v)
        pltpu.sync_copy(tab_hbm, tab_v)

        def start_in(c):
            b = c % 2
            off = base + c * chunk
            return (
                pltpu.async_copy(ii_hbm.at[pl.ds(off, chunk)], ii_v.at[b],
                                 sems_in[b]),
                pltpu.async_copy(jj_hbm.at[pl.ds(off, chunk)], jj_v.at[b],
                                 sems_in[b]),
            )

        in_flight = {0: start_in(0)}
        out_flight = {}
        for c in range(n_chunks):
            b = c % 2
            if c + 1 < n_chunks:
                in_flight[c + 1] = start_in(c + 1)
            for cp in in_flight.pop(c):
                cp.wait()
            if c >= 2:
                out_flight.pop(c - 2).wait()

            def vec_body(t):
                ii = ii_v[b, pl.ds(t, _LANES)]
                jj = jj_v[b, pl.ds(t, _LANES)]
                zi = plsc.load_gather(z_v, [ii])
                zj = plsc.load_gather(z_v, [jj])
                xi = plsc.load_gather(tab_v, [zi])
                xj = plsc.load_gather(tab_v, [zj])
                w_v[b, pl.ds(t, _LANES)] = xi * xj

            plsc.parallel_loop(0, chunk, _LANES, unroll=5)(vec_body)

            off = base + c * chunk
            out_flight[c] = pltpu.async_copy(
                w_v.at[b], w_hbm.at[pl.ds(off, chunk)], sems_out[b])
        for cp in out_flight.values():
            cp.wait()

    return body(z, idx_i, idx_j, table_pad)


def _tc_body(rx_ref, ry_ref, rz_ref, w_ref, out_ref):
    rx = rx_ref[...].reshape(1, -1)
    ry = ry_ref[...].reshape(1, -1)
    rz = rz_ref[...].reshape(1, -1)
    w = w_ref[...].reshape(1, -1)
    d2 = rx * rx + ry * ry + rz * rz
    inv_d = lax.rsqrt(d2)
    x = d2 * inv_d * (jnp.pi / _R_CUT)
    two_c = 2.0 * jnp.cos(x)
    t1 = jnp.sin(x) * (w * (jnp.sqrt(2.0 / _R_CUT)) * inv_d)
    rbf_rows = []
    s_prev, s_cur = jnp.zeros_like(t1), t1
    for _ in range(_N_RBF):
        rbf_rows.append(s_cur)
        s_prev, s_cur = s_cur, two_c * s_cur - s_prev
    rbf = jnp.concatenate(rbf_rows, axis=0)
    c1d = _C1 * inv_d
    out_ref[...] = jnp.concatenate(
        [_C0 * rbf, (c1d * ry) * rbf, (c1d * rz) * rbf, (c1d * rx) * rbf],
        axis=0,
    )


def kernel(z, idx_i, idx_j, r_ij, embed_table):
    n_edges = idx_i.shape[0]
    n_nodes = z.shape[0]
    table_flat = embed_table.reshape(-1)
    pad = (-table_flat.shape[0]) % 128
    table_pad = jnp.pad(table_flat, (0, pad))

    w = _sc_gather_w(z.astype(jnp.int32), idx_i.astype(jnp.int32),
                     idx_j.astype(jnp.int32), table_pad, n_nodes, n_edges)

    bl = 32768
    grid = -(-n_edges // bl)
    in_spec = pl.BlockSpec((bl,), lambda i: (i,))
    out_t = pl.pallas_call(
        _tc_body,
        grid=(grid,),
        in_specs=[in_spec] * 4,
        out_specs=pl.BlockSpec((32, bl), lambda i: (0, i)),
        out_shape=jax.ShapeDtypeStruct((32, n_edges), jnp.float32),
        compiler_params=pltpu.CompilerParams(
            dimension_semantics=("arbitrary",),
        ),
    )(r_ij[:, 0], r_ij[:, 1], r_ij[:, 2], w)
    return out_t.T


# speedup vs baseline: 3.0676x; 1.0581x over previous
import functools

import jax
import jax.numpy as jnp
from jax import lax
from jax.experimental import pallas as pl
from jax.experimental.pallas import tpu as pltpu
from jax.experimental.pallas import tpu_sc as plsc

_N_RBF = 8
_R_CUT = 5.0
_C0 = 0.28209479177387814
_C1 = 0.4886025119029199

_LANES = 16
_SC_CHUNK = 2000


def _sc_gather_w(z, idx_i, idx_j, table_pad, n_nodes, n_edges):
    info = plsc.get_sparse_core_info()
    nc, ns = info.num_cores, info.num_subcores
    nw = nc * ns
    per_tile = n_edges // nw
    chunk = _SC_CHUNK
    n_chunks = per_tile // chunk
    mesh = plsc.VectorSubcoreMesh(core_axis_name="c", subcore_axis_name="s")

    @functools.partial(
        pl.kernel,
        mesh=mesh,
        out_type=jax.ShapeDtypeStruct((n_edges,), jnp.float32),
        scratch_types=[
            pltpu.VMEM((n_nodes,), jnp.int32),
            pltpu.VMEM((table_pad.shape[0],), jnp.float32),
            pltpu.VMEM((2, chunk), jnp.int32),
            pltpu.VMEM((2, chunk), jnp.int32),
            pltpu.VMEM((2, chunk), jnp.float32),
            pltpu.SemaphoreType.DMA,
            pltpu.SemaphoreType.DMA,
            pltpu.SemaphoreType.DMA,
            pltpu.SemaphoreType.DMA,
        ],
        compiler_params=pltpu.CompilerParams(
            needs_layout_passes=False,
            use_tc_tiling_on_sc=False,
        ),
    )
    def body(z_hbm, ii_hbm, jj_hbm, tab_hbm, w_hbm, z_v, tab_v, ii_v, jj_v,
             w_v, sem_i0, sem_i1, sem_o0, sem_o1):
        wid = lax.axis_index("s") * nc + lax.axis_index("c")
        base = wid * per_tile
        sems_in = (sem_i0, sem_i1)
        sems_out = (sem_o0, sem_o1)
        pltpu.sync_copy(z_hbm, z_v)
        pltpu.sync_copy(tab_hbm, tab_v)

        def start_in(c):
            b = c % 2
            off = base + c * chunk
            return (
                pltpu.async_copy(ii_hbm.at[pl.ds(off, chunk)], ii_v.at[b],
                                 sems_in[b]),
                pltpu.async_copy(jj_hbm.at[pl.ds(off, chunk)], jj_v.at[b],
                                 sems_in[b]),
            )

        in_flight = {0: start_in(0)}
        out_flight = {}
        for c in range(n_chunks):
            b = c % 2
            if c + 1 < n_chunks:
                in_flight[c + 1] = start_in(c + 1)
            for cp in in_flight.pop(c):
                cp.wait()
            if c >= 2:
                out_flight.pop(c - 2).wait()

            def vec_body(t):
                ii = ii_v[b, pl.ds(t, _LANES)]
                jj = jj_v[b, pl.ds(t, _LANES)]
                zi = plsc.load_gather(z_v, [ii])
                zj = plsc.load_gather(z_v, [jj])
                xi = plsc.load_gather(tab_v, [zi])
                xj = plsc.load_gather(tab_v, [zj])
                w_v[b, pl.ds(t, _LANES)] = xi * xj

            plsc.parallel_loop(0, chunk, _LANES, unroll=5)(vec_body)

            off = base + c * chunk
            out_flight[c] = pltpu.async_copy(
                w_v.at[b], w_hbm.at[pl.ds(off, chunk)], sems_out[b])
        for cp in out_flight.values():
            cp.wait()

    return body(z, idx_i, idx_j, table_pad)


def _tc_body(rx_ref, ry_ref, rz_ref, w_ref, out_ref):
    rx = rx_ref[...].reshape(1, -1)
    ry = ry_ref[...].reshape(1, -1)
    rz = rz_ref[...].reshape(1, -1)
    w = w_ref[...].reshape(1, -1)
    d2 = rx * rx + ry * ry + rz * rz
    inv_d = lax.rsqrt(d2)
    x = d2 * inv_d * (jnp.pi / _R_CUT)
    two_c = 2.0 * jnp.cos(x)
    t1 = jnp.sin(x) * (w * (jnp.sqrt(2.0 / _R_CUT)) * inv_d)
    rbf_rows = []
    s_prev, s_cur = jnp.zeros_like(t1), t1
    for _ in range(_N_RBF):
        rbf_rows.append(s_cur)
        s_prev, s_cur = s_cur, two_c * s_cur - s_prev
    rbf = jnp.concatenate(rbf_rows, axis=0)
    c1d = _C1 * inv_d
    out_ref[...] = jnp.concatenate(
        [_C0 * rbf, (c1d * ry) * rbf, (c1d * rz) * rbf, (c1d * rx) * rbf],
        axis=0,
    )


def kernel(z, idx_i, idx_j, r_ij, embed_table):
    n_edges = idx_i.shape[0]
    n_nodes = z.shape[0]
    table_flat = embed_table.reshape(-1)
    pad = (-table_flat.shape[0]) % 128
    table_pad = jnp.pad(table_flat, (0, pad))

    w = _sc_gather_w(z.astype(jnp.int32), idx_i.astype(jnp.int32),
                     idx_j.astype(jnp.int32), table_pad, n_nodes, n_edges)

    bl = 65536
    grid = -(-n_edges // bl)
    in_spec = pl.BlockSpec((bl,), lambda i: (i,))
    out_t = pl.pallas_call(
        _tc_body,
        grid=(grid,),
        in_specs=[in_spec] * 4,
        out_specs=pl.BlockSpec((32, bl), lambda i: (0, i)),
        out_shape=jax.ShapeDtypeStruct((32, n_edges), jnp.float32),
        compiler_params=pltpu.CompilerParams(
            dimension_semantics=("arbitrary",),
        ),
    )(r_ij[:, 0], r_ij[:, 1], r_ij[:, 2], w)
    return out_t.T
